# baseline (speedup 1.0000x reference)
import functools

import jax
import jax.numpy as jnp
from jax import lax
from jax.experimental import pallas as pl
from jax.experimental.pallas import tpu as pltpu
from jax.experimental.pallas import tpu_sc as plsc

EMBED = 128
H1 = 512
H2 = 512
OUT = 256
BATCH = 4096

_BB = 1024


def _sc_gather(table, idx, batch):
    info = plsc.get_sparse_core_info()
    ncores = info.num_cores
    nw = ncores * info.num_subcores
    b_per_w = batch // nw
    mesh = plsc.VectorSubcoreMesh(
        core_axis_name="c", subcore_axis_name="s", num_cores=ncores
    )

    half = b_per_w // 2

    @functools.partial(
        pl.kernel,
        mesh=mesh,
        out_type=jax.ShapeDtypeStruct((batch, EMBED), jnp.float32),
        scratch_types=[
            pltpu.VMEM((half,), jnp.int32),
            pltpu.VMEM((half,), jnp.int32),
            pltpu.VMEM((half, EMBED), jnp.float32),
            pltpu.VMEM((half, EMBED), jnp.float32),
            pltpu.SemaphoreType.DMA,
            pltpu.SemaphoreType.DMA,
            pltpu.SemaphoreType.DMA,
            pltpu.SemaphoreType.DMA,
        ],
    )
    def gather_kernel(
        table_hbm, idx_hbm, out_hbm,
        idx_v0, idx_v1, rows_v0, rows_v1, g0, g1, s0, s1,
    ):
        wid = lax.axis_index("s") * ncores + lax.axis_index("c")
        base = wid * b_per_w
        pltpu.sync_copy(idx_hbm.at[pl.ds(base, half)], idx_v0)
        c0 = pltpu.async_copy(table_hbm.at[idx_v0], rows_v0, g0)
        pltpu.sync_copy(idx_hbm.at[pl.ds(base + half, half)], idx_v1)
        c1 = pltpu.async_copy(table_hbm.at[idx_v1], rows_v1, g1)
        c0.wait()
        st0 = pltpu.async_copy(rows_v0, out_hbm.at[pl.ds(base, half)], s0)
        c1.wait()
        st1 = pltpu.async_copy(rows_v1, out_hbm.at[pl.ds(base + half, half)], s1)
        st0.wait()
        st1.wait()

    return gather_kernel(table, idx)


def _fcnn_body(x_ref, w1_ref, b1_ref, w2_ref, b2_ref, w3_ref, b3_ref, o_ref):
    x = x_ref[...].astype(jnp.bfloat16)
    h = jnp.dot(x, w1_ref[...], preferred_element_type=jnp.float32)
    h = jnp.maximum(h + b1_ref[...], 0.0).astype(jnp.bfloat16)
    h = jnp.dot(h, w2_ref[...], preferred_element_type=jnp.float32)
    h = jnp.maximum(h + b2_ref[...], 0.0).astype(jnp.bfloat16)
    o_ref[...] = (
        jnp.dot(h, w3_ref[...], preferred_element_type=jnp.float32) + b3_ref[...]
    )


def _fcnn(x, W1, b1, W2, b2, W3, b3):
    batch = x.shape[0]
    grid = (batch // _BB,)
    return pl.pallas_call(
        _fcnn_body,
        grid=grid,
        in_specs=[
            pl.BlockSpec((_BB, EMBED), lambda i: (i, 0)),
            pl.BlockSpec((EMBED, H1), lambda i: (0, 0)),
            pl.BlockSpec((1, H1), lambda i: (0, 0)),
            pl.BlockSpec((H1, H2), lambda i: (0, 0)),
            pl.BlockSpec((1, H2), lambda i: (0, 0)),
            pl.BlockSpec((H2, OUT), lambda i: (0, 0)),
            pl.BlockSpec((1, OUT), lambda i: (0, 0)),
        ],
        out_specs=pl.BlockSpec((_BB, OUT), lambda i: (i, 0)),
        out_shape=jax.ShapeDtypeStruct((batch, OUT), jnp.float32),
    )(
        x,
        W1.astype(jnp.bfloat16),
        b1.reshape(1, H1),
        W2.astype(jnp.bfloat16),
        b2.reshape(1, H2),
        W3.astype(jnp.bfloat16),
        b3.reshape(1, OUT),
    )


def kernel(protein_features, table, W1, b1, W2, b2, W3, b3):
    idx = protein_features.astype(jnp.int32)
    x = _sc_gather(table, idx, BATCH)
    return _fcnn(x, W1, b1, W2, b2, W3, b3)


# speedup vs baseline: 1.0446x; 1.0446x over previous
import functools

import jax
import jax.numpy as jnp
from jax import lax
from jax.experimental import pallas as pl
from jax.experimental.pallas import tpu as pltpu
from jax.experimental.pallas import tpu_sc as plsc

EMBED = 128
H1 = 512
H2 = 512
OUT = 256
BATCH = 4096

_BB = 2048


def _sc_gather(table, idx, batch):
    info = plsc.get_sparse_core_info()
    ncores = info.num_cores
    nw = ncores * info.num_subcores
    b_per_w = batch // nw
    mesh = plsc.VectorSubcoreMesh(
        core_axis_name="c", subcore_axis_name="s", num_cores=ncores
    )

    half = b_per_w // 2

    @functools.partial(
        pl.kernel,
        mesh=mesh,
        out_type=jax.ShapeDtypeStruct((batch, EMBED), jnp.float32),
        scratch_types=[
            pltpu.VMEM((half,), jnp.int32),
            pltpu.VMEM((half,), jnp.int32),
            pltpu.VMEM((half, EMBED), jnp.float32),
            pltpu.VMEM((half, EMBED), jnp.float32),
            pltpu.SemaphoreType.DMA,
            pltpu.SemaphoreType.DMA,
            pltpu.SemaphoreType.DMA,
            pltpu.SemaphoreType.DMA,
        ],
    )
    def gather_kernel(
        table_hbm, idx_hbm, out_hbm,
        idx_v0, idx_v1, rows_v0, rows_v1, g0, g1, s0, s1,
    ):
        wid = lax.axis_index("s") * ncores + lax.axis_index("c")
        base = wid * b_per_w
        pltpu.sync_copy(idx_hbm.at[pl.ds(base, half)], idx_v0)
        c0 = pltpu.async_copy(table_hbm.at[idx_v0], rows_v0, g0)
        pltpu.sync_copy(idx_hbm.at[pl.ds(base + half, half)], idx_v1)
        c1 = pltpu.async_copy(table_hbm.at[idx_v1], rows_v1, g1)
        c0.wait()
        st0 = pltpu.async_copy(rows_v0, out_hbm.at[pl.ds(base, half)], s0)
        c1.wait()
        st1 = pltpu.async_copy(rows_v1, out_hbm.at[pl.ds(base + half, half)], s1)
        st0.wait()
        st1.wait()

    return gather_kernel(table, idx)


def _fcnn_body(x_ref, w1_ref, b1_ref, w2_ref, b2_ref, w3_ref, b3_ref, o_ref):
    x = x_ref[...].astype(jnp.bfloat16)
    h = jnp.dot(x, w1_ref[...], preferred_element_type=jnp.float32)
    h = jnp.maximum(h + b1_ref[...], 0.0).astype(jnp.bfloat16)
    h = jnp.dot(h, w2_ref[...], preferred_element_type=jnp.float32)
    h = jnp.maximum(h + b2_ref[...], 0.0).astype(jnp.bfloat16)
    o_ref[...] = (
        jnp.dot(h, w3_ref[...], preferred_element_type=jnp.float32) + b3_ref[...]
    )


def _fcnn(x, W1, b1, W2, b2, W3, b3):
    batch = x.shape[0]
    grid = (batch // _BB,)
    return pl.pallas_call(
        _fcnn_body,
        grid=grid,
        in_specs=[
            pl.BlockSpec((_BB, EMBED), lambda i: (i, 0)),
            pl.BlockSpec((EMBED, H1), lambda i: (0, 0)),
            pl.BlockSpec((1, H1), lambda i: (0, 0)),
            pl.BlockSpec((H1, H2), lambda i: (0, 0)),
            pl.BlockSpec((1, H2), lambda i: (0, 0)),
            pl.BlockSpec((H2, OUT), lambda i: (0, 0)),
            pl.BlockSpec((1, OUT), lambda i: (0, 0)),
        ],
        out_specs=pl.BlockSpec((_BB, OUT), lambda i: (i, 0)),
        out_shape=jax.ShapeDtypeStruct((batch, OUT), jnp.float32),
    )(
        x,
        W1.astype(jnp.bfloat16),
        b1.reshape(1, H1),
        W2.astype(jnp.bfloat16),
        b2.reshape(1, H2),
        W3.astype(jnp.bfloat16),
        b3.reshape(1, OUT),
    )


def kernel(protein_features, table, W1, b1, W2, b2, W3, b3):
    idx = protein_features.astype(jnp.int32)
    x = _sc_gather(table, idx, BATCH)
    return _fcnn(x, W1, b1, W2, b2, W3, b3)
